# baseline (speedup 1.0000x reference)
import functools

import jax
import jax.numpy as jnp
import numpy as np
from jax import lax
from jax.experimental import pallas as pl
from jax.experimental.pallas import tpu as pltpu
from jax.experimental.pallas import tpu_sc as plsc

B = 64
V = 1_000_000

R_SC = 16
R_TC = B - R_SC
BV = 4096
NB = (V + BV - 1) // BV
TAIL = V - (NB - 1) * BV
NCHUNK = BV // 128

BV2 = 8192
NB2 = (V + BV2 - 1) // BV2
TAIL2 = V - (NB2 - 1) * BV2
NCHUNK2 = BV2 // 128

SC_NC = 2
SC_NS = 16
SC_NW = SC_NC * SC_NS
SC_ELEMS = R_SC * V
SC_PER_W = SC_ELEMS // SC_NW
SC_CHUNK = 10_000
SC_VPC = SC_CHUNK // 16
SC_NCHUNKS = SC_PER_W // SC_CHUNK
SC_BASE = R_TC * V

_INV_TEMP = np.float32(1.25)
_TINY = np.float32(np.finfo(np.float32).tiny)
_K1 = np.uint32(0)
_K2 = np.uint32(42)
_KS2 = np.uint32(int(_K1) ^ int(_K2) ^ 0x1BD11BDA)
_ROTS = ((13, 15, 26, 6), (17, 29, 16, 24))
_KS = (_K1, _K2, _KS2)
_INTMAX = np.int32(0x7FFFFFFF)


def _threefry_bits(x1):
    x0 = x1
    x1 = ((x1 << np.uint32(13)) | (x1 >> np.uint32(19))) ^ x0
    for d in _ROTS[0][1:]:
        x0 = x0 + x1
        x1 = ((x1 << np.uint32(d)) | (x1 >> np.uint32(32 - d))) ^ x0
    x0 = x0 + _KS[1]
    x1 = x1 + np.uint32((int(_KS[2]) + 1) & 0xFFFFFFFF)
    for i in range(1, 5):
        for d in _ROTS[i % 2]:
            x0 = x0 + x1
            x1 = ((x1 << np.uint32(d)) | (x1 >> np.uint32(32 - d))) ^ x0
        x0 = x0 + _KS[(i + 1) % 3]
        x1 = x1 + np.uint32((int(_KS[(i + 2) % 3]) + i + 1) & 0xFFFFFFFF)
    return x0 ^ x1


def _gumbel_from_bits(bits):
    fbits = (bits >> np.uint32(9)) | np.uint32(0x3F800000)
    floats = jax.lax.bitcast_convert_type(fbits, jnp.float32) - np.float32(1.0)
    u = floats + _TINY
    return -jnp.log(-jnp.log(u))


def _resolve(pmax, pidx, nrows, row_base, out_ref):
    m = jnp.max(pmax, axis=1, keepdims=True)
    pidx_s = jax.lax.bitcast_convert_type(pidx, jnp.int32)
    minx1 = jnp.min(jnp.where(pmax == m, pidx_s, _INTMAX), axis=1, keepdims=True)
    rowv = (
        jax.lax.broadcasted_iota(jnp.int32, (nrows, 1), 0) + np.int32(row_base)
    ) * np.int32(V)
    out_ref[...] = minx1 - rowv - np.int32(42)


def _main_kernel(scores_ref, out_ref, pmax_ref, pidx_ref):
    j = pl.program_id(0)

    @pl.when(j == 0)
    def _init():
        pmax_ref[...] = jnp.full((R_TC, 128), -jnp.inf, jnp.float32)
        pidx_ref[...] = jnp.zeros((R_TC, 128), jnp.uint32)

    @pl.when(j == NB - 1)
    def _mask_tail():
        scores_ref[:, TAIL:] = jnp.full((R_TC, BV - TAIL), -jnp.inf, jnp.float32)

    lane = jax.lax.broadcasted_iota(jnp.uint32, (R_TC, 128), 1)
    row_off = jax.lax.broadcasted_iota(jnp.uint32, (R_TC, 128), 0) * np.uint32(V)
    x1_base = (row_off + lane) + (j * BV + 42).astype(jnp.uint32)

    pmax = pmax_ref[...]
    pidx = pidx_ref[...]
    for c in range(NCHUNK):
        x1 = x1_base + np.uint32(c * 128) if c else x1_base
        g = _gumbel_from_bits(_threefry_bits(x1))
        val = scores_ref[:, c * 128 : (c + 1) * 128] * _INV_TEMP + g
        gt = val > pmax
        pmax = jnp.maximum(pmax, val)
        pidx = jnp.where(gt, x1, pidx)
    pmax_ref[...] = pmax
    pidx_ref[...] = pidx

    @pl.when(j == NB - 1)
    def _done():
        _resolve(pmax_ref[...], pidx_ref[...], R_TC, 0, out_ref)


def _tail_kernel(scores_ref, bits_ref, out_ref, pmax_ref, pidx_ref):
    j = pl.program_id(0)

    @pl.when(j == 0)
    def _init():
        pmax_ref[...] = jnp.full((R_SC, 128), -jnp.inf, jnp.float32)
        pidx_ref[...] = jnp.zeros((R_SC, 128), jnp.uint32)

    @pl.when(j == NB2 - 1)
    def _mask_tail():
        scores_ref[:, TAIL2:] = jnp.full(
            (R_SC, BV2 - TAIL2), -jnp.inf, jnp.float32
        )

    lane = jax.lax.broadcasted_iota(jnp.uint32, (R_SC, 128), 1)
    row_off = (
        jax.lax.broadcasted_iota(jnp.uint32, (R_SC, 128), 0) + np.uint32(R_TC)
    ) * np.uint32(V)
    x1_base = (row_off + lane) + (j * BV2 + 42).astype(jnp.uint32)

    pmax = pmax_ref[...]
    pidx = pidx_ref[...]
    for c in range(NCHUNK2):
        x1 = x1_base + np.uint32(c * 128) if c else x1_base
        g = _gumbel_from_bits(bits_ref[:, c * 128 : (c + 1) * 128])
        val = scores_ref[:, c * 128 : (c + 1) * 128] * _INV_TEMP + g
        gt = val > pmax
        pmax = jnp.maximum(pmax, val)
        pidx = jnp.where(gt, x1, pidx)
    pmax_ref[...] = pmax
    pidx_ref[...] = pidx

    @pl.when(j == NB2 - 1)
    def _done():
        _resolve(pmax_ref[...], pidx_ref[...], R_SC, R_TC, out_ref)


def _sc_bits_kernel(out_hbm, buf):
    cid = lax.axis_index("c")
    sid = lax.axis_index("s")
    wid = sid * SC_NC + cid
    base = wid * SC_PER_W
    iota16 = jax.lax.broadcasted_iota(jnp.uint32, (16,), 0)

    def chunk_body(c, carry):
        start = (base + c * SC_CHUNK + (SC_BASE + 42)).astype(jnp.uint32)
        x1_0 = iota16 + start

        def vec_body(i, x1):
            buf[pl.ds(i * 16, 16)] = _threefry_bits(x1)
            return x1 + np.uint32(16)

        lax.fori_loop(0, SC_VPC, vec_body, x1_0, unroll=False)
        pltpu.sync_copy(buf, out_hbm.at[pl.ds(base + c * SC_CHUNK, SC_CHUNK)])
        return carry

    lax.fori_loop(0, SC_NCHUNKS, chunk_body, jnp.int32(0), unroll=False)


@jax.jit
def kernel(scores):
    bits = pl.kernel(
        _sc_bits_kernel,
        mesh=plsc.VectorSubcoreMesh(core_axis_name="c", subcore_axis_name="s"),
        out_type=jax.ShapeDtypeStruct((SC_ELEMS,), jnp.uint32),
        scratch_types=[pltpu.VMEM((SC_CHUNK,), jnp.uint32)],
    )()

    out_a = pl.pallas_call(
        _main_kernel,
        grid=(NB,),
        in_specs=[pl.BlockSpec((R_TC, BV), lambda j: (0, j))],
        out_specs=pl.BlockSpec((R_TC, 1), lambda j: (0, 0)),
        out_shape=jax.ShapeDtypeStruct((R_TC, 1), jnp.int32),
        scratch_shapes=[
            pltpu.VMEM((R_TC, 128), jnp.float32),
            pltpu.VMEM((R_TC, 128), jnp.uint32),
        ],
    )(scores)

    out_b = pl.pallas_call(
        _tail_kernel,
        grid=(NB2,),
        in_specs=[
            pl.BlockSpec((R_SC, BV2), lambda j: (R_TC // R_SC, j)),
            pl.BlockSpec((R_SC, BV2), lambda j: (0, j)),
        ],
        out_specs=pl.BlockSpec((R_SC, 1), lambda j: (0, 0)),
        out_shape=jax.ShapeDtypeStruct((R_SC, 1), jnp.int32),
        scratch_shapes=[
            pltpu.VMEM((R_SC, 128), jnp.float32),
            pltpu.VMEM((R_SC, 128), jnp.uint32),
        ],
    )(scores, bits.reshape(R_SC, V))

    return jnp.concatenate([out_a.reshape(R_TC), out_b.reshape(R_SC)])


# speedup vs baseline: 2.3967x; 2.3967x over previous
import jax
import jax.numpy as jnp
import numpy as np
from jax.experimental import pallas as pl
from jax.experimental.pallas import tpu as pltpu

B = 64
V = 1_000_000
BV = 5120
NB = (V + BV - 1) // BV
TAIL = V - (NB - 1) * BV
NCHUNK = BV // 128

_TEMP = np.float32(0.8)
_INV_TEMP = np.float32(1.25)
_TINY = np.float32(np.finfo(np.float32).tiny)
_K1 = np.uint32(0)
_K2 = np.uint32(42)
_KS2 = np.uint32(int(_K1) ^ int(_K2) ^ 0x1BD11BDA)
_ROTS = ((13, 15, 26, 6), (17, 29, 16, 24))
_KS = (_K1, _K2, _KS2)
_INTMAX = np.int32(0x7FFFFFFF)


def _threefry_bits(x1):
    x0 = x1
    x1 = ((x1 << np.uint32(13)) | (x1 >> np.uint32(19))) ^ x0
    for d in _ROTS[0][1:]:
        x0 = x0 + x1
        x1 = ((x1 << np.uint32(d)) | (x1 >> np.uint32(32 - d))) ^ x0
    x0 = x0 + _KS[1]
    x1 = x1 + np.uint32((int(_KS[2]) + 1) & 0xFFFFFFFF)
    for i in range(1, 5):
        for d in _ROTS[i % 2]:
            x0 = x0 + x1
            x1 = ((x1 << np.uint32(d)) | (x1 >> np.uint32(32 - d))) ^ x0
        x0 = x0 + _KS[(i + 1) % 3]
        x1 = x1 + np.uint32((int(_KS[(i + 2) % 3]) + i + 1) & 0xFFFFFFFF)
    return x0 ^ x1


def _sampler_kernel(scores_ref, out_ref, pmax_ref, pidx_ref):
    j = pl.program_id(0)

    @pl.when(j == 0)
    def _init():
        pmax_ref[...] = jnp.full((B, 128), -jnp.inf, jnp.float32)
        pidx_ref[...] = jnp.zeros((B, 128), jnp.uint32)

    @pl.when(j == NB - 1)
    def _mask_tail():
        scores_ref[:, TAIL:] = jnp.full((B, BV - TAIL), -jnp.inf, jnp.float32)

    lane = jax.lax.broadcasted_iota(jnp.uint32, (B, 128), 1)
    row_off = jax.lax.broadcasted_iota(jnp.uint32, (B, 128), 0) * np.uint32(V)
    x1_base = (row_off + lane) + (j * BV + 42).astype(jnp.uint32)

    pmax = pmax_ref[...]
    pidx = pidx_ref[...]
    for c in range(NCHUNK):
        x1 = x1_base + np.uint32(c * 128) if c else x1_base
        bits = _threefry_bits(x1)
        fbits = (bits >> np.uint32(9)) | np.uint32(0x3F800000)
        floats = jax.lax.bitcast_convert_type(fbits, jnp.float32) - np.float32(1.0)
        u = floats + _TINY
        g = -jnp.log(-jnp.log(u))
        val = scores_ref[:, c * 128 : (c + 1) * 128] * _INV_TEMP + g
        gt = val > pmax
        pmax = jnp.maximum(pmax, val)
        pidx = jnp.where(gt, x1, pidx)
    pmax_ref[...] = pmax
    pidx_ref[...] = pidx

    @pl.when(j == NB - 1)
    def _done():
        pmax = pmax_ref[...]
        pidx = pidx_ref[...]
        m = jnp.max(pmax, axis=1, keepdims=True)
        pidx_s = jax.lax.bitcast_convert_type(pidx, jnp.int32)
        minx1 = jnp.min(
            jnp.where(pmax == m, pidx_s, _INTMAX), axis=1, keepdims=True
        )
        rowv = jax.lax.broadcasted_iota(jnp.int32, (B, 1), 0) * np.int32(V)
        out_ref[...] = minx1 - rowv - np.int32(42)


@jax.jit
def kernel(scores):
    out = pl.pallas_call(
        _sampler_kernel,
        grid=(NB,),
        in_specs=[pl.BlockSpec((B, BV), lambda j: (0, j))],
        out_specs=pl.BlockSpec((B, 1), lambda j: (0, 0)),
        out_shape=jax.ShapeDtypeStruct((B, 1), jnp.int32),
        scratch_shapes=[
            pltpu.VMEM((B, 128), jnp.float32),
            pltpu.VMEM((B, 128), jnp.uint32),
        ],
    )(scores)
    return out.reshape(B)


# speedup vs baseline: 2.4213x; 1.0102x over previous
import jax
import jax.numpy as jnp
import numpy as np
from jax.experimental import pallas as pl
from jax.experimental.pallas import tpu as pltpu

B = 64
V = 1_000_000
BV = 3072
NB = (V + BV - 1) // BV
TAIL = V - (NB - 1) * BV
NCHUNK = BV // 128

_TEMP = np.float32(0.8)
_INV_TEMP = np.float32(1.25)
_TINY = np.float32(np.finfo(np.float32).tiny)
_K1 = np.uint32(0)
_K2 = np.uint32(42)
_KS2 = np.uint32(int(_K1) ^ int(_K2) ^ 0x1BD11BDA)
_ROTS = ((13, 15, 26, 6), (17, 29, 16, 24))
_KS = (_K1, _K2, _KS2)
_INTMAX = np.int32(0x7FFFFFFF)


def _threefry_bits(x1):
    x0 = x1
    x1 = ((x1 << np.uint32(13)) | (x1 >> np.uint32(19))) ^ x0
    for d in _ROTS[0][1:]:
        x0 = x0 + x1
        x1 = ((x1 << np.uint32(d)) | (x1 >> np.uint32(32 - d))) ^ x0
    x0 = x0 + _KS[1]
    x1 = x1 + np.uint32((int(_KS[2]) + 1) & 0xFFFFFFFF)
    for i in range(1, 5):
        for d in _ROTS[i % 2]:
            x0 = x0 + x1
            x1 = ((x1 << np.uint32(d)) | (x1 >> np.uint32(32 - d))) ^ x0
        x0 = x0 + _KS[(i + 1) % 3]
        x1 = x1 + np.uint32((int(_KS[(i + 2) % 3]) + i + 1) & 0xFFFFFFFF)
    return x0 ^ x1


def _sampler_kernel(scores_ref, out_ref, pmax_ref, pidx_ref):
    j = pl.program_id(0)

    @pl.when(j == 0)
    def _init():
        pmax_ref[...] = jnp.full((B, 128), -jnp.inf, jnp.float32)
        pidx_ref[...] = jnp.zeros((B, 128), jnp.uint32)

    @pl.when(j == NB - 1)
    def _mask_tail():
        scores_ref[:, TAIL:] = jnp.full((B, BV - TAIL), -jnp.inf, jnp.float32)

    lane = jax.lax.broadcasted_iota(jnp.uint32, (B, 128), 1)
    row_off = jax.lax.broadcasted_iota(jnp.uint32, (B, 128), 0) * np.uint32(V)
    x1_base = (row_off + lane) + (j * BV + 42).astype(jnp.uint32)

    pmax = pmax_ref[...]
    pidx = pidx_ref[...]
    for c in range(NCHUNK):
        x1 = x1_base + np.uint32(c * 128) if c else x1_base
        bits = _threefry_bits(x1)
        fbits = (bits >> np.uint32(9)) | np.uint32(0x3F800000)
        floats = jax.lax.bitcast_convert_type(fbits, jnp.float32) - np.float32(1.0)
        u = floats + _TINY
        g = -jnp.log(-jnp.log(u))
        val = scores_ref[:, c * 128 : (c + 1) * 128] * _INV_TEMP + g
        gt = val > pmax
        pmax = jnp.maximum(pmax, val)
        pidx = jnp.where(gt, x1, pidx)
    pmax_ref[...] = pmax
    pidx_ref[...] = pidx

    @pl.when(j == NB - 1)
    def _done():
        pmax = pmax_ref[...]
        pidx = pidx_ref[...]
        m = jnp.max(pmax, axis=1, keepdims=True)
        pidx_s = jax.lax.bitcast_convert_type(pidx, jnp.int32)
        minx1 = jnp.min(
            jnp.where(pmax == m, pidx_s, _INTMAX), axis=1, keepdims=True
        )
        rowv = jax.lax.broadcasted_iota(jnp.int32, (B, 1), 0) * np.int32(V)
        out_ref[...] = minx1 - rowv - np.int32(42)


@jax.jit
def kernel(scores):
    out = pl.pallas_call(
        _sampler_kernel,
        grid=(NB,),
        in_specs=[pl.BlockSpec((B, BV), lambda j: (0, j))],
        out_specs=pl.BlockSpec((B, 1), lambda j: (0, 0)),
        out_shape=jax.ShapeDtypeStruct((B, 1), jnp.int32),
        scratch_shapes=[
            pltpu.VMEM((B, 128), jnp.float32),
            pltpu.VMEM((B, 128), jnp.uint32),
        ],
    )(scores)
    return out.reshape(B)
